# baseline (speedup 1.0000x reference)
import functools

import jax
import jax.numpy as jnp
from jax import lax
from jax.experimental import pallas as pl
from jax.experimental.pallas import tpu as pltpu
from jax.experimental.pallas import tpu_sc as plsc

EMB_DIM = 64
BCHUNK = 128


@functools.cache
def _make_gather(batch: int, n_fields: int, n_emb: int):
  NC, NS = 2, 16
  NW = NC * NS
  assert batch % BCHUNK == 0
  blocks_per_field = batch // BCHUNK
  n_chunks = n_fields * blocks_per_field
  assert n_chunks % NW == 0
  ch_per_w = n_chunks // NW
  assert ch_per_w % 4 == 0

  mesh = plsc.VectorSubcoreMesh(core_axis_name="c", subcore_axis_name="s")

  @functools.partial(
      pl.kernel,
      out_type=jax.ShapeDtypeStruct((n_fields, EMB_DIM, batch), jnp.bfloat16),
      mesh=mesh,
      scratch_types=[
          pltpu.VMEM((ch_per_w, BCHUNK), jnp.int32),
          pltpu.VMEM((4, BCHUNK, EMB_DIM), jnp.float32),
          pltpu.VMEM((2, EMB_DIM, BCHUNK), jnp.bfloat16),
          pltpu.SemaphoreType.DMA((4,)),
          pltpu.SemaphoreType.DMA((2,)),
      ],
      compiler_params=pltpu.CompilerParams(
          use_tc_tiling_on_sc=False, needs_layout_passes=False
      ),
  )
  def grab(idx_hbm, table_hbm, out_hbm, idx_v, rows_v, obuf_v, gsem, osem):
    wid = lax.axis_index("s") * NC + lax.axis_index("c")
    base_chunk = wid * ch_per_w
    pltpu.sync_copy(idx_hbm.at[pl.ds(base_chunk, ch_per_w)], idx_v)

    def gather(c, p):
      return pltpu.make_async_copy(
          table_hbm.at[idx_v.at[c]], rows_v.at[p], gsem.at[p]
      )

    def store(c, q):
      ci = base_chunk + c
      f = ci // blocks_per_field
      b0 = (ci % blocks_per_field) * BCHUNK
      return pltpu.make_async_copy(
          obuf_v.at[q], out_hbm.at[f, :, pl.ds(b0, BCHUNK)], osem.at[q]
      )

    iota16 = lax.iota(jnp.int32, 16)
    r_even = [iota16 * 2 + 32 * g for g in range(4)]
    r_odd = [iota16 * 2 + 32 * g + 1 for g in range(4)]

    gather(0, 0).start()
    gather(1, 1).start()

    @pl.loop(0, ch_per_w, step=4)
    def _(c0):
      for p in range(4):
        c = c0 + p
        q = p % 2
        gather(c, p).wait()

        @pl.when(c + 2 < ch_per_w)
        def _():
          gather(c + 2, (p + 2) % 4).start()

        @pl.when(c >= 2)
        def _():
          store(c - 2, q).wait()

        src = rows_v.at[p]
        dst = obuf_v.at[q]

        @pl.loop(0, EMB_DIM, unroll=4)
        def _(j):
          jj = jnp.full((16,), j, jnp.int32)
          for g in range(4):
            va = plsc.load_gather(src, [r_even[g], jj])
            vb = plsc.load_gather(src, [r_odd[g], jj])
            dst[j, pl.ds(32 * g, 32)] = plsc.pack(
                va, vb, format=plsc.PackFormat.INTERLEAVED
            )

        store(c, q).start()

    store(ch_per_w - 2, 0).wait()
    store(ch_per_w - 1, 1).wait()

  return grab


def kernel(input, embedding_weight):
  b, f = input.shape
  idx = input.astype(jnp.int32).T.reshape(f * (b // BCHUNK), BCHUNK)
  grab = _make_gather(b, f, embedding_weight.shape[0])
  out_t = grab(idx, embedding_weight)
  return out_t.transpose(2, 0, 1)


# speedup vs baseline: 1.1455x; 1.1455x over previous
import functools

import jax
import jax.numpy as jnp
from jax import lax
from jax.experimental import pallas as pl
from jax.experimental.pallas import tpu as pltpu
from jax.experimental.pallas import tpu_sc as plsc

EMB_DIM = 64
BPC = 4


@functools.cache
def _make_gather(batch: int, n_fields: int, n_emb: int):
  NC, NS = 2, 16
  NW = NC * NS
  chunk = BPC * n_fields
  assert chunk <= 128
  n_chunks = batch // BPC
  assert batch % (BPC * NW) == 0
  ch_per_w = n_chunks // NW
  assert ch_per_w % 4 == 0

  mesh = plsc.VectorSubcoreMesh(core_axis_name="c", subcore_axis_name="s")

  @functools.partial(
      pl.kernel,
      out_type=jax.ShapeDtypeStruct(
          (n_chunks, BPC * n_fields, EMB_DIM), jnp.bfloat16
      ),
      mesh=mesh,
      scratch_types=[
          pltpu.VMEM((ch_per_w, chunk), jnp.int32),
          pltpu.VMEM((4, chunk, EMB_DIM), jnp.bfloat16),
          pltpu.SemaphoreType.DMA((4,)),
          pltpu.SemaphoreType.DMA((4,)),
      ],
      compiler_params=pltpu.CompilerParams(use_tc_tiling_on_sc=False),
  )
  def grab(idx_hbm, table_hbm, out_hbm, idx_v, rows_v, gsem, osem):
    wid = lax.axis_index("s") * NC + lax.axis_index("c")
    base_chunk = wid * ch_per_w
    pltpu.sync_copy(idx_hbm.at[pl.ds(base_chunk, ch_per_w)], idx_v)

    def gather(c, p):
      return pltpu.make_async_copy(
          table_hbm.at[idx_v.at[c]], rows_v.at[p], gsem.at[p]
      )

    def store(c, p):
      return pltpu.make_async_copy(
          rows_v.at[p], out_hbm.at[base_chunk + c], osem.at[p]
      )

    gather(0, 0).start()
    gather(1, 1).start()

    @pl.loop(0, ch_per_w, step=4)
    def _(c0):
      for p in range(4):
        c = c0 + p
        gather(c, p).wait()

        @pl.when(c >= 2)
        def _():
          store(c - 2, (p + 2) % 4).wait()

        store(c, p).start()

        @pl.when(c + 2 < ch_per_w)
        def _():
          gather(c + 2, (p + 2) % 4).start()

    store(ch_per_w - 2, (ch_per_w - 2) % 4).wait()
    store(ch_per_w - 1, (ch_per_w - 1) % 4).wait()

  return grab


def kernel(input, embedding_weight):
  b, f = input.shape
  idx = input.astype(jnp.int32).reshape(b // BPC, BPC * f)
  wbf = embedding_weight.T.astype(jnp.bfloat16).T
  grab = _make_gather(b, f, embedding_weight.shape[0])
  return grab(idx, wbf).reshape(b, f, EMB_DIM)


# speedup vs baseline: 1.1786x; 1.0289x over previous
import functools

import jax
import jax.numpy as jnp
from jax import lax
from jax.experimental import pallas as pl
from jax.experimental.pallas import tpu as pltpu
from jax.experimental.pallas import tpu_sc as plsc

EMB_DIM = 64
CHUNK = 128


@functools.cache
def _make_gather(n_rows: int, n_emb: int):
  NC, NS = 2, 16
  NW = NC * NS
  assert n_rows % (NW * CHUNK) == 0
  ch_per_w = n_rows // (NW * CHUNK)
  assert ch_per_w % 4 == 0

  mesh = plsc.VectorSubcoreMesh(core_axis_name="c", subcore_axis_name="s")

  @functools.partial(
      pl.kernel,
      out_type=jax.ShapeDtypeStruct((n_rows, EMB_DIM), jnp.bfloat16),
      mesh=mesh,
      scratch_types=[
          pltpu.VMEM((ch_per_w, CHUNK), jnp.int32),
          pltpu.VMEM((4, CHUNK, EMB_DIM), jnp.float32),
          pltpu.VMEM((2, CHUNK, EMB_DIM), jnp.bfloat16),
          pltpu.SemaphoreType.DMA((4,)),
          pltpu.SemaphoreType.DMA((2,)),
      ],
      compiler_params=pltpu.CompilerParams(use_tc_tiling_on_sc=False),
  )
  def grab(idx_hbm, table_hbm, out_hbm, idx_v, rows_v, bfout_v, gsem, osem):
    wid = lax.axis_index("s") * NC + lax.axis_index("c")
    base_chunk = wid * ch_per_w
    pltpu.sync_copy(idx_hbm.at[pl.ds(base_chunk, ch_per_w)], idx_v)

    def gather(c, p):
      return pltpu.make_async_copy(
          table_hbm.at[idx_v.at[c]], rows_v.at[p], gsem.at[p]
      )

    def store(c, q):
      return pltpu.make_async_copy(
          bfout_v.at[q],
          out_hbm.at[pl.ds((base_chunk + c) * CHUNK, CHUNK)],
          osem.at[q],
      )

    gather(0, 0).start()
    gather(1, 1).start()

    @pl.loop(0, ch_per_w, step=4)
    def _(c0):
      for p in range(4):
        c = c0 + p
        q = p % 2
        gather(c, p).wait()

        @pl.when(c + 2 < ch_per_w)
        def _():
          gather(c + 2, (p + 2) % 4).start()

        @pl.when(c >= 2)
        def _():
          store(c - 2, q).wait()

        src = rows_v.at[p]
        dst = bfout_v.at[q]

        @pl.loop(0, CHUNK, unroll=4)
        def _(r):
          for h in range(4):
            a = src[r, pl.ds(h * 16, 16)].astype(jnp.bfloat16)
            dst[r, pl.ds(h * 16, 16)] = a

        store(c, q).start()

    store(ch_per_w - 2, 0).wait()
    store(ch_per_w - 1, 1).wait()

  return grab


def kernel(input, embedding_weight):
  b, f = input.shape
  n_rows = b * f
  idx = input.astype(jnp.int32).reshape(n_rows // CHUNK, CHUNK)
  grab = _make_gather(n_rows, embedding_weight.shape[0])
  out = grab(idx, embedding_weight)
  return out.reshape(b, f, EMB_DIM)


# speedup vs baseline: 1.3306x; 1.1290x over previous
import functools

import jax
import jax.numpy as jnp
from jax import lax
from jax.experimental import pallas as pl
from jax.experimental.pallas import tpu as pltpu
from jax.experimental.pallas import tpu_sc as plsc

EMB_DIM = 64
CHUNK = 128


@functools.cache
def _make_gather(n_rows: int, n_emb: int):
  NC, NS = 2, 16
  NW = NC * NS
  assert n_rows % (NW * CHUNK) == 0
  ch_per_w = n_rows // (NW * CHUNK)
  assert ch_per_w % 4 == 0

  mesh = plsc.VectorSubcoreMesh(core_axis_name="c", subcore_axis_name="s")

  @functools.partial(
      pl.kernel,
      out_type=jax.ShapeDtypeStruct((n_rows, EMB_DIM), jnp.float32),
      mesh=mesh,
      scratch_types=[
          pltpu.VMEM((ch_per_w, CHUNK), jnp.int32),
          pltpu.VMEM((4, CHUNK, EMB_DIM), jnp.float32),
          pltpu.SemaphoreType.DMA((4,)),
          pltpu.SemaphoreType.DMA((4,)),
      ],
      compiler_params=pltpu.CompilerParams(use_tc_tiling_on_sc=False),
  )
  def grab(idx_hbm, table_hbm, out_hbm, idx_v, rows_v, gsem, osem):
    wid = lax.axis_index("s") * NC + lax.axis_index("c")
    base_chunk = wid * ch_per_w
    pltpu.sync_copy(idx_hbm.at[pl.ds(base_chunk, ch_per_w)], idx_v)

    def gather(c, p):
      return pltpu.make_async_copy(
          table_hbm.at[idx_v.at[c]], rows_v.at[p], gsem.at[p]
      )

    def store(c, p):
      return pltpu.make_async_copy(
          rows_v.at[p],
          out_hbm.at[pl.ds((base_chunk + c) * CHUNK, CHUNK)],
          osem.at[p],
      )

    gather(0, 0).start()
    gather(1, 1).start()

    @pl.loop(0, ch_per_w, step=4)
    def _(c0):
      for p in range(4):
        c = c0 + p
        gather(c, p).wait()

        @pl.when(c >= 2)
        def _():
          store(c - 2, (p + 2) % 4).wait()

        store(c, p).start()

        @pl.when(c + 2 < ch_per_w)
        def _():
          gather(c + 2, (p + 2) % 4).start()

    store(ch_per_w - 2, (ch_per_w - 2) % 4).wait()
    store(ch_per_w - 1, (ch_per_w - 1) % 4).wait()

  return grab


def kernel(input, embedding_weight):
  b, f = input.shape
  n_rows = b * f
  idx = input.astype(jnp.int32).reshape(n_rows // CHUNK, CHUNK)
  grab = _make_gather(n_rows, embedding_weight.shape[0])
  out = grab(idx, embedding_weight)
  return out.astype(jnp.bfloat16).reshape(b, f, EMB_DIM)
